# baseline (speedup 1.0000x reference)
import functools

import jax
import jax.numpy as jnp
from jax import lax
from jax.experimental import pallas as pl
from jax.experimental.pallas import tpu as pltpu
from jax.experimental.pallas import tpu_sc as plsc

B = 16384
D = 32
L = 16
NC, NS = 2, 16
NW = NC * NS
BPW = B // NW
V = 1000000
TPW = 244 * 128
SCAN = TPW * NW
TAIL = V - SCAN
CK = 512
NCHK = TPW // CK
CAP = 1024
NG = 4
PB = B + L
ROWW = 128

_mesh = plsc.VectorSubcoreMesh(core_axis_name="c", subcore_axis_name="s")

_SCAN_SCRATCH = [
    pltpu.VMEM((B,), jnp.int32),
    pltpu.VMEM((B,), jnp.int32),
    pltpu.VMEM((B,), jnp.int32),
    pltpu.VMEM((D, CK), jnp.float32),
    pltpu.VMEM((D, CK), jnp.float32),
    pltpu.VMEM((D, TAIL), jnp.float32),
    pltpu.VMEM((CAP,), jnp.int32),
    pltpu.VMEM((CAP,), jnp.int32),
    pltpu.VMEM((CAP,), jnp.int32),
    pltpu.VMEM((CAP,), jnp.int32),
    pltpu.VMEM((CAP,), jnp.int32),
    pltpu.VMEM((CAP,), jnp.int32),
    pltpu.VMEM((NG * L,), jnp.int32),
    pltpu.VMEM((NG * L,), jnp.int32),
    pltpu.VMEM((NG, L), jnp.int32),
    pltpu.VMEM((L, ROWW), jnp.float32),
    pltpu.SemaphoreType.DMA,
    pltpu.SemaphoreType.DMA,
    pltpu.SemaphoreType.DMA,
]


@functools.partial(
    pl.kernel,
    mesh=_mesh,
    compiler_params=pltpu.CompilerParams(needs_layout_passes=False),
    out_type=(
        jax.ShapeDtypeStruct((PB, ROWW), jnp.float32),
        jax.ShapeDtypeStruct((PB, ROWW), jnp.float32),
        jax.ShapeDtypeStruct((PB, ROWW), jnp.float32),
    ),
    scratch_types=_SCAN_SCRATCH,
)
def _scan_kernel(users_hbm, pos_hbm, neg_hbm, utab, itab, utail, itail,
                 uemb, pemb, nemb,
                 ulist, plist, nlist, buf0, buf1, tbuf,
                 mu, mbu, mp, mbp, mn, mbn, clr, clbf, clb, stag,
                 sem0, sem1, sems):
    wid = lax.axis_index("s") * NC + lax.axis_index("c")
    lo = wid * TPW
    hi = jnp.where(wid == NW - 1, V, lo + TPW)

    pltpu.sync_copy(users_hbm, ulist)
    pltpu.sync_copy(pos_hbm, plist)
    pltpu.sync_copy(neg_hbm, nlist)

    def fill(i, carry):
        neg1 = jnp.full((L,), -1, jnp.int32)
        for ref in (mu, mp, mn):
            ref[pl.ds(i * L, L)] = neg1
        return carry

    lax.fori_loop(0, CAP // L, fill, 0)

    def scan_list(list_ref, midx, mb):
        def body(g, off):
            v = list_ref[pl.ds(g * L, L)]
            m = (v >= lo) & (v < hi)
            bvec = g * L + lax.iota(jnp.int32, L)
            plsc.store_compressed(midx.at[pl.ds(off, L)], v, mask=m)
            plsc.store_compressed(mb.at[pl.ds(off, L)], bvec, mask=m)
            return off + plsc.all_reduce_population_count(m)[0]

        return lax.fori_loop(0, B // L, body, 0)

    ucnt = scan_list(ulist, mu, mbu)
    pcnt = scan_list(plist, mp, mbp)
    ncnt = scan_list(nlist, mn, mbn)

    def process(buf, c0, size, midx, mb, mcnt, dst):
        def sel(g, off):
            v = midx[pl.ds(g * L, L)]
            m = (v >= c0) & (v < c0 + size)
            bvec = mb[pl.ds(g * L, L)]
            plsc.store_compressed(clr.at[pl.ds(off, L)], v - c0, mask=m)
            plsc.store_compressed(clbf.at[pl.ds(off, L)], bvec, mask=m)
            return off + plsc.all_reduce_population_count(m)[0]

        for g in range(NG):
            clr[pl.ds(g * L, L)] = jnp.zeros((L,), jnp.int32)
            clbf[pl.ds(g * L, L)] = B + lax.iota(jnp.int32, L)
        cnt = lax.fori_loop(0, (mcnt + L - 1) // L, sel, 0)
        ngroups = (cnt + L - 1) // L

        def extract(g, carry):
            rvec = clr[pl.ds(g * L, L)]
            clb[g, pl.ds(0, L)] = clbf[pl.ds(g * L, L)]
            lane = lax.iota(jnp.int32, L)
            for c in range(D):
                val = plsc.load_gather(buf, [jnp.full((L,), c, jnp.int32),
                                             rvec])
                plsc.store_scatter(stag, [lane, jnp.full((L,), c, jnp.int32)],
                                   val)
            pltpu.async_copy(stag, dst.at[clb.at[g]], sems).wait()
            return carry

        lax.fori_loop(0, ngroups, extract, 0)

    bufs = (buf0, buf1)
    semc = (sem0, sem1)

    def scan_table(tab, jobs):
        pltpu.async_copy(tab.at[:, pl.ds(pl.multiple_of(lo, 128), CK)],
                         buf0, sem0)
        pltpu.async_copy(tab.at[:, pl.ds(pl.multiple_of(lo + CK, 128), CK)],
                         buf1, sem1)

        def chunkpair(cp, carry):
            for par in range(2):
                c = 2 * cp + par
                pltpu.make_async_copy(tab.at[:, pl.ds(0, CK)], bufs[par],
                                      semc[par]).wait()
                for midx, mb, mcnt, dst in jobs:
                    process(bufs[par], lo + c * CK, CK, midx, mb, mcnt, dst)
                nxt = c + 2

                @pl.when(nxt < NCHK)
                def _():
                    r0 = pl.multiple_of(lo + nxt * CK, 128)
                    pltpu.async_copy(tab.at[:, pl.ds(r0, CK)], bufs[par],
                                     semc[par])
            return carry

        lax.fori_loop(0, NCHK // 2, chunkpair, 0)
        pltpu.make_async_copy(tab.at[:, pl.ds(0, CK)], buf0, sem0).wait()
        for midx, mb, mcnt, dst in jobs:
            process(buf0, lo + (NCHK - 1) * CK, CK, midx, mb, mcnt, dst)

    scan_table(utab, ((mu, mbu, ucnt, uemb),))
    scan_table(itab, ((mp, mbp, pcnt, pemb), (mn, mbn, ncnt, nemb)))

    @pl.when(wid == NW - 1)
    def _():
        pltpu.sync_copy(utail, tbuf)
        process(tbuf, SCAN, TAIL, mu, mbu, ucnt, uemb)
        pltpu.sync_copy(itail, tbuf)
        process(tbuf, SCAN, TAIL, mp, mbp, pcnt, pemb)
        process(tbuf, SCAN, TAIL, mn, mbn, ncnt, nemb)


@functools.partial(
    pl.kernel,
    mesh=_mesh,
    compiler_params=pltpu.CompilerParams(needs_layout_passes=False),
    out_type=(
        jax.ShapeDtypeStruct((B,), jnp.float32),
        jax.ShapeDtypeStruct((B,), jnp.float32),
    ),
    scratch_types=[
        pltpu.VMEM((BPW // 4, ROWW), jnp.float32),
        pltpu.VMEM((BPW // 4, ROWW), jnp.float32),
        pltpu.VMEM((BPW // 4, ROWW), jnp.float32),
        pltpu.VMEM((BPW,), jnp.float32),
        pltpu.VMEM((BPW,), jnp.float32),
        pltpu.SemaphoreType.DMA,
    ],
)
def _dot_kernel(uemb, pemb, nemb, pout_hbm, nout_hbm,
                ubuf, pbuf, nbuf, pout, nout, sem):
    wid = lax.axis_index("s") * NC + lax.axis_index("c")
    base = wid * BPW
    SUB = BPW // 4

    def subchunk(s, carry):
        b0 = base + s * SUB
        cu = pltpu.async_copy(uemb.at[pl.ds(b0, SUB)], ubuf, sem)
        cp = pltpu.async_copy(pemb.at[pl.ds(b0, SUB)], pbuf, sem)
        cn = pltpu.async_copy(nemb.at[pl.ds(b0, SUB)], nbuf, sem)
        cu.wait(); cp.wait(); cn.wait()

        def group(g, carry2):
            row0 = g * L
            ridx = row0 + lax.iota(jnp.int32, L)
            accp = jnp.zeros((L,), jnp.float32)
            accn = jnp.zeros((L,), jnp.float32)
            for d in range(D):
                cidx = jnp.full((L,), d, jnp.int32)
                uv = plsc.load_gather(ubuf, [ridx, cidx])
                pv = plsc.load_gather(pbuf, [ridx, cidx])
                nv = plsc.load_gather(nbuf, [ridx, cidx])
                accp = accp + uv * pv
                accn = accn + uv * nv
            pout[pl.ds(s * SUB + row0, L)] = accp
            nout[pl.ds(s * SUB + row0, L)] = accn
            return carry2

        lax.fori_loop(0, SUB // L, group, 0)
        return carry

    lax.fori_loop(0, 4, subchunk, 0)
    pltpu.sync_copy(pout, pout_hbm.at[pl.ds(base, BPW)])
    pltpu.sync_copy(nout, nout_hbm.at[pl.ds(base, BPW)])


def kernel(users, positive_items, negative_items, user_table, item_table):
    u = users.astype(jnp.int32)
    p = positive_items.astype(jnp.int32)
    n = negative_items.astype(jnp.int32)
    ut = user_table.T
    it = item_table.T
    utail = ut[:, SCAN:]
    itail = it[:, SCAN:]
    uemb, pemb, nemb = _scan_kernel(u, p, n, ut, it, utail, itail)
    return _dot_kernel(uemb, pemb, nemb)


# speedup vs baseline: 1.0092x; 1.0092x over previous
import functools

import jax
import jax.numpy as jnp
from jax import lax
from jax.experimental import pallas as pl
from jax.experimental.pallas import tpu as pltpu
from jax.experimental.pallas import tpu_sc as plsc

B = 16384
D = 32
L = 16
NC, NS = 2, 16
NW = NC * NS
BPW = B // NW
V = 1000000
TPW = 244 * 128
SCAN = TPW * NW
TAIL = V - SCAN
CK = 512
NCHK = TPW // CK
CAP = 1024
NG = 4
PB = B + L
ROWW = 128

_mesh = plsc.VectorSubcoreMesh(core_axis_name="c", subcore_axis_name="s")

_SCAN_SCRATCH = [
    pltpu.VMEM((B,), jnp.int32),
    pltpu.VMEM((B,), jnp.int32),
    pltpu.VMEM((B,), jnp.int32),
    pltpu.VMEM((D, CK), jnp.float32),
    pltpu.VMEM((D, CK), jnp.float32),
    pltpu.VMEM((D, TAIL), jnp.float32),
    pltpu.VMEM((CAP,), jnp.int32),
    pltpu.VMEM((CAP,), jnp.int32),
    pltpu.VMEM((CAP,), jnp.int32),
    pltpu.VMEM((CAP,), jnp.int32),
    pltpu.VMEM((CAP,), jnp.int32),
    pltpu.VMEM((CAP,), jnp.int32),
    pltpu.VMEM((256,), jnp.int32),
    pltpu.VMEM((256,), jnp.int32),
    pltpu.VMEM((256,), jnp.int32),
    pltpu.VMEM((256,), jnp.int32),
    pltpu.VMEM((NG * L,), jnp.int32),
    pltpu.VMEM((NG * L,), jnp.int32),
    pltpu.VMEM((NG, L), jnp.int32),
    pltpu.VMEM((L, ROWW), jnp.float32),
    pltpu.SemaphoreType.DMA,
    pltpu.SemaphoreType.DMA,
    pltpu.SemaphoreType.DMA,
]


@functools.partial(
    pl.kernel,
    mesh=_mesh,
    compiler_params=pltpu.CompilerParams(needs_layout_passes=False),
    out_type=(
        jax.ShapeDtypeStruct((PB, ROWW), jnp.float32),
        jax.ShapeDtypeStruct((PB, ROWW), jnp.float32),
        jax.ShapeDtypeStruct((PB, ROWW), jnp.float32),
    ),
    scratch_types=_SCAN_SCRATCH,
)
def _scan_kernel(users_hbm, pos_hbm, neg_hbm, utab, itab, utail, itail,
                 uemb, pemb, nemb,
                 ulist, plist, nlist, buf0, buf1, tbuf,
                 mu, mbu, mp, mbp, mn, mbn, s0i, s0b, s1i, s1b,
                 clr, clbf, clb, stag,
                 sem0, sem1, sems):
    wid = lax.axis_index("s") * NC + lax.axis_index("c")
    lo = wid * TPW
    hi = jnp.where(wid == NW - 1, V, lo + TPW)

    pltpu.sync_copy(users_hbm, ulist)
    pltpu.sync_copy(pos_hbm, plist)
    pltpu.sync_copy(neg_hbm, nlist)

    def fill(i, carry):
        neg1 = jnp.full((L,), -1, jnp.int32)
        for ref in (mu, mp, mn):
            ref[pl.ds(i * L, L)] = neg1
        return carry

    lax.fori_loop(0, CAP // L, fill, 0)

    def scan_list(list_ref, midx, mb):
        def body(g4, off):
            for k in range(4):
                g = g4 * 4 + k
                v = list_ref[pl.ds(g * L, L)]
                m = (v >= lo) & (v < hi)
                bvec = g * L + lax.iota(jnp.int32, L)
                plsc.store_compressed(midx.at[pl.ds(off, L)], v, mask=m)
                plsc.store_compressed(mb.at[pl.ds(off, L)], bvec, mask=m)
                off = off + plsc.all_reduce_population_count(m)[0]
            return off

        return lax.fori_loop(0, B // L // 4, body, 0)

    ucnt = scan_list(ulist, mu, mbu)
    pcnt = scan_list(plist, mp, mbp)
    ncnt = scan_list(nlist, mn, mbn)

    def process(buf, c0, size, midx, mb, mcnt, dst):
        def sel(g, off):
            v = midx[pl.ds(g * L, L)]
            m = (v >= c0) & (v < c0 + size)
            bvec = mb[pl.ds(g * L, L)]
            plsc.store_compressed(clr.at[pl.ds(off, L)], v - c0, mask=m)
            plsc.store_compressed(clbf.at[pl.ds(off, L)], bvec, mask=m)
            return off + plsc.all_reduce_population_count(m)[0]

        for g in range(NG):
            clr[pl.ds(g * L, L)] = jnp.zeros((L,), jnp.int32)
            clbf[pl.ds(g * L, L)] = B + lax.iota(jnp.int32, L)
        cnt = lax.fori_loop(0, (mcnt + L - 1) // L, sel, 0)
        ngroups = (cnt + L - 1) // L

        def extract(g, carry):
            rvec = clr[pl.ds(g * L, L)]
            clb[g, pl.ds(0, L)] = clbf[pl.ds(g * L, L)]
            lane = lax.iota(jnp.int32, L)
            for c in range(D):
                val = plsc.load_gather(buf, [jnp.full((L,), c, jnp.int32),
                                             rvec])
                plsc.store_scatter(stag, [lane, jnp.full((L,), c, jnp.int32)],
                                   val)
            pltpu.async_copy(stag, dst.at[clb.at[g]], sems).wait()
            return carry

        lax.fori_loop(0, ngroups, extract, 0)

    bufs = (buf0, buf1)
    semc = (sem0, sem1)

    SUPC = 8
    SUPR = SUPC * CK
    SUBCAP = 256

    def build_sublist(midx, mb, mcnt, sidx, sb, s0):
        for g in range(SUBCAP // L):
            sidx[pl.ds(g * L, L)] = jnp.full((L,), -1, jnp.int32)

        def selsup(g4, off):
            for k in range(4):
                g = g4 * 4 + k
                v = midx[pl.ds(g * L, L)]
                m = (v >= s0) & (v < s0 + SUPR)
                bv = mb[pl.ds(g * L, L)]
                plsc.store_compressed(sidx.at[pl.ds(off, L)], v, mask=m)
                plsc.store_compressed(sb.at[pl.ds(off, L)], bv, mask=m)
                off = off + plsc.all_reduce_population_count(m)[0]
            return off

        return lax.fori_loop(0, (mcnt + 4 * L - 1) // (4 * L), selsup, 0)

    def scan_table(tab, jobs):
        pltpu.async_copy(tab.at[:, pl.ds(pl.multiple_of(lo, 128), CK)],
                         buf0, sem0)
        pltpu.async_copy(tab.at[:, pl.ds(pl.multiple_of(lo + CK, 128), CK)],
                         buf1, sem1)

        def chunkpair(cp, carry):
            subcnts = carry
            for par in range(2):
                c = 2 * cp + par
                s0 = lo + (c // SUPC) * SUPR

                def rebuild():
                    return tuple(
                        build_sublist(midx, mb, mcnt, sidx, sb, s0)
                        for midx, mb, mcnt, _, sidx, sb in jobs)

                def keep():
                    return subcnts

                subcnts = lax.cond(c % SUPC == 0, rebuild, keep)
                pltpu.make_async_copy(tab.at[:, pl.ds(0, CK)], bufs[par],
                                      semc[par]).wait()
                for j, (_, _, _, dst, sidx, sb) in enumerate(jobs):
                    process(bufs[par], lo + c * CK, CK, sidx, sb,
                            subcnts[j], dst)
                nxt = c + 2

                @pl.when(nxt < NCHK)
                def _():
                    r0 = pl.multiple_of(lo + nxt * CK, 128)
                    pltpu.async_copy(tab.at[:, pl.ds(r0, CK)], bufs[par],
                                     semc[par])
            return subcnts

        subcnts = lax.fori_loop(0, NCHK // 2, chunkpair,
                                (jnp.int32(0),) * len(jobs))
        pltpu.make_async_copy(tab.at[:, pl.ds(0, CK)], buf0, sem0).wait()
        for j, (_, _, _, dst, sidx, sb) in enumerate(jobs):
            process(buf0, lo + (NCHK - 1) * CK, CK, sidx, sb,
                    subcnts[j], dst)

    scan_table(utab, ((mu, mbu, ucnt, uemb, s0i, s0b),))
    scan_table(itab, ((mp, mbp, pcnt, pemb, s0i, s0b),
                      (mn, mbn, ncnt, nemb, s1i, s1b)))

    @pl.when(wid == NW - 1)
    def _():
        pltpu.sync_copy(utail, tbuf)
        process(tbuf, SCAN, TAIL, mu, mbu, ucnt, uemb)
        pltpu.sync_copy(itail, tbuf)
        process(tbuf, SCAN, TAIL, mp, mbp, pcnt, pemb)
        process(tbuf, SCAN, TAIL, mn, mbn, ncnt, nemb)


@functools.partial(
    pl.kernel,
    mesh=_mesh,
    compiler_params=pltpu.CompilerParams(needs_layout_passes=False),
    out_type=(
        jax.ShapeDtypeStruct((B,), jnp.float32),
        jax.ShapeDtypeStruct((B,), jnp.float32),
    ),
    scratch_types=[
        pltpu.VMEM((BPW // 4, ROWW), jnp.float32),
        pltpu.VMEM((BPW // 4, ROWW), jnp.float32),
        pltpu.VMEM((BPW // 4, ROWW), jnp.float32),
        pltpu.VMEM((BPW,), jnp.float32),
        pltpu.VMEM((BPW,), jnp.float32),
        pltpu.SemaphoreType.DMA,
    ],
)
def _dot_kernel(uemb, pemb, nemb, pout_hbm, nout_hbm,
                ubuf, pbuf, nbuf, pout, nout, sem):
    wid = lax.axis_index("s") * NC + lax.axis_index("c")
    base = wid * BPW
    SUB = BPW // 4

    def subchunk(s, carry):
        b0 = base + s * SUB
        cu = pltpu.async_copy(uemb.at[pl.ds(b0, SUB)], ubuf, sem)
        cp = pltpu.async_copy(pemb.at[pl.ds(b0, SUB)], pbuf, sem)
        cn = pltpu.async_copy(nemb.at[pl.ds(b0, SUB)], nbuf, sem)
        cu.wait(); cp.wait(); cn.wait()

        def group(g, carry2):
            row0 = g * L
            ridx = row0 + lax.iota(jnp.int32, L)
            accp = jnp.zeros((L,), jnp.float32)
            accn = jnp.zeros((L,), jnp.float32)
            for d in range(D):
                cidx = jnp.full((L,), d, jnp.int32)
                uv = plsc.load_gather(ubuf, [ridx, cidx])
                pv = plsc.load_gather(pbuf, [ridx, cidx])
                nv = plsc.load_gather(nbuf, [ridx, cidx])
                accp = accp + uv * pv
                accn = accn + uv * nv
            pout[pl.ds(s * SUB + row0, L)] = accp
            nout[pl.ds(s * SUB + row0, L)] = accn
            return carry2

        lax.fori_loop(0, SUB // L, group, 0)
        return carry

    lax.fori_loop(0, 4, subchunk, 0)
    pltpu.sync_copy(pout, pout_hbm.at[pl.ds(base, BPW)])
    pltpu.sync_copy(nout, nout_hbm.at[pl.ds(base, BPW)])


def kernel(users, positive_items, negative_items, user_table, item_table):
    u = users.astype(jnp.int32)
    p = positive_items.astype(jnp.int32)
    n = negative_items.astype(jnp.int32)
    ut = user_table.T
    it = item_table.T
    utail = ut[:, SCAN:]
    itail = it[:, SCAN:]
    uemb, pemb, nemb = _scan_kernel(u, p, n, ut, it, utail, itail)
    return _dot_kernel(uemb, pemb, nemb)


# speedup vs baseline: 1.0174x; 1.0082x over previous
import functools

import jax
import jax.numpy as jnp
from jax import lax
from jax.experimental import pallas as pl
from jax.experimental.pallas import tpu as pltpu
from jax.experimental.pallas import tpu_sc as plsc

B = 16384
D = 32
L = 16
NC, NS = 2, 16
NW = NC * NS
BPW = B // NW
V = 1000000
TPW = 244 * 128
SCAN = TPW * NW
TAIL = V - SCAN
CK = 512
NCHK = TPW // CK
CAP = 1024
NG = 4
PB = B + L
ROWW = 128

_mesh = plsc.VectorSubcoreMesh(core_axis_name="c", subcore_axis_name="s")

_SCAN_SCRATCH = [
    pltpu.VMEM((B,), jnp.int32),
    pltpu.VMEM((B,), jnp.int32),
    pltpu.VMEM((B,), jnp.int32),
    pltpu.VMEM((D, CK), jnp.float32),
    pltpu.VMEM((D, CK), jnp.float32),
    pltpu.VMEM((D, TAIL), jnp.float32),
    pltpu.VMEM((CAP,), jnp.int32),
    pltpu.VMEM((CAP,), jnp.int32),
    pltpu.VMEM((CAP,), jnp.int32),
    pltpu.VMEM((CAP,), jnp.int32),
    pltpu.VMEM((CAP,), jnp.int32),
    pltpu.VMEM((CAP,), jnp.int32),
    pltpu.VMEM((256,), jnp.int32),
    pltpu.VMEM((256,), jnp.int32),
    pltpu.VMEM((256,), jnp.int32),
    pltpu.VMEM((256,), jnp.int32),
    pltpu.VMEM((NG * L,), jnp.int32),
    pltpu.VMEM((NG * L,), jnp.int32),
    pltpu.VMEM((NG, L), jnp.int32),
    pltpu.VMEM((NG * L, ROWW), jnp.float32),
    pltpu.SemaphoreType.DMA,
    pltpu.SemaphoreType.DMA,
    pltpu.SemaphoreType.DMA,
]


@functools.partial(
    pl.kernel,
    mesh=_mesh,
    compiler_params=pltpu.CompilerParams(needs_layout_passes=False),
    out_type=(
        jax.ShapeDtypeStruct((PB, ROWW), jnp.float32),
        jax.ShapeDtypeStruct((PB, ROWW), jnp.float32),
        jax.ShapeDtypeStruct((PB, ROWW), jnp.float32),
    ),
    scratch_types=_SCAN_SCRATCH,
)
def _scan_kernel(users_hbm, pos_hbm, neg_hbm, utab, itab, utail, itail,
                 uemb, pemb, nemb,
                 ulist, plist, nlist, buf0, buf1, tbuf,
                 mu, mbu, mp, mbp, mn, mbn, s0i, s0b, s1i, s1b,
                 clr, clbf, clb, stag,
                 sem0, sem1, sems):
    wid = lax.axis_index("s") * NC + lax.axis_index("c")
    lo = wid * TPW
    hi = jnp.where(wid == NW - 1, V, lo + TPW)

    pltpu.sync_copy(users_hbm, ulist)
    pltpu.sync_copy(pos_hbm, plist)
    pltpu.sync_copy(neg_hbm, nlist)

    def fill(i, carry):
        neg1 = jnp.full((L,), -1, jnp.int32)
        for ref in (mu, mp, mn):
            ref[pl.ds(i * L, L)] = neg1
        return carry

    lax.fori_loop(0, CAP // L, fill, 0)

    def scan_list(list_ref, midx, mb):
        def body(g4, off):
            for k in range(4):
                g = g4 * 4 + k
                v = list_ref[pl.ds(g * L, L)]
                m = (v >= lo) & (v < hi)
                bvec = g * L + lax.iota(jnp.int32, L)
                plsc.store_compressed(midx.at[pl.ds(off, L)], v, mask=m)
                plsc.store_compressed(mb.at[pl.ds(off, L)], bvec, mask=m)
                off = off + plsc.all_reduce_population_count(m)[0]
            return off

        return lax.fori_loop(0, B // L // 4, body, 0)

    ucnt = scan_list(ulist, mu, mbu)
    pcnt = scan_list(plist, mp, mbp)
    ncnt = scan_list(nlist, mn, mbn)

    GB = L * ROWW * 4

    def process(buf, c0, size, midx, mb, mcnt, dst, pending):
        def sel(g, off):
            v = midx[pl.ds(g * L, L)]
            m = (v >= c0) & (v < c0 + size)
            bvec = mb[pl.ds(g * L, L)]
            plsc.store_compressed(clr.at[pl.ds(off, L)], v - c0, mask=m)
            plsc.store_compressed(clbf.at[pl.ds(off, L)], bvec, mask=m)
            return off + plsc.all_reduce_population_count(m)[0]

        for g in range(NG):
            clr[pl.ds(g * L, L)] = jnp.zeros((L,), jnp.int32)
            clbf[pl.ds(g * L, L)] = B + lax.iota(jnp.int32, L)
        cnt = lax.fori_loop(0, (mcnt + L - 1) // L, sel, 0)
        ngroups = (cnt + L - 1) // L

        def drain(i, c2):
            pltpu.make_async_copy(stag.at[pl.ds(0, L)],
                                  dst.at[pl.ds(B, L)], sems).wait()
            return c2

        lax.fori_loop(0, pending, drain, 0)

        def extract(g, carry):
            rvec = clr[pl.ds(g * L, L)]
            clb[g, pl.ds(0, L)] = clbf[pl.ds(g * L, L)]
            lane = lax.iota(jnp.int32, L)
            for c in range(D):
                val = plsc.load_gather(buf, [jnp.full((L,), c, jnp.int32),
                                             rvec])
                plsc.store_scatter(
                    stag.at[pl.ds(g * L, L)],
                    [lane, jnp.full((L,), c, jnp.int32)], val)
            pltpu.async_copy(stag.at[pl.ds(g * L, L)], dst.at[clb.at[g]],
                             sems)
            return carry

        lax.fori_loop(0, ngroups, extract, 0)
        return ngroups

    bufs = (buf0, buf1)
    semc = (sem0, sem1)

    SUPC = 8
    SUPR = SUPC * CK
    SUBCAP = 256

    def build_sublist(midx, mb, mcnt, sidx, sb, s0):
        for g in range(SUBCAP // L):
            sidx[pl.ds(g * L, L)] = jnp.full((L,), -1, jnp.int32)

        def selsup(g4, off):
            for k in range(4):
                g = g4 * 4 + k
                v = midx[pl.ds(g * L, L)]
                m = (v >= s0) & (v < s0 + SUPR)
                bv = mb[pl.ds(g * L, L)]
                plsc.store_compressed(sidx.at[pl.ds(off, L)], v, mask=m)
                plsc.store_compressed(sb.at[pl.ds(off, L)], bv, mask=m)
                off = off + plsc.all_reduce_population_count(m)[0]
            return off

        return lax.fori_loop(0, (mcnt + 4 * L - 1) // (4 * L), selsup, 0)

    def scan_table(tab, jobs, pend0):
        pltpu.async_copy(tab.at[:, pl.ds(pl.multiple_of(lo, 128), CK)],
                         buf0, sem0)
        pltpu.async_copy(tab.at[:, pl.ds(pl.multiple_of(lo + CK, 128), CK)],
                         buf1, sem1)

        def chunkpair(cp, carry):
            *subcnts, pending = carry
            subcnts = tuple(subcnts)
            for par in range(2):
                c = 2 * cp + par
                s0 = lo + (c // SUPC) * SUPR

                def rebuild():
                    return tuple(
                        build_sublist(midx, mb, mcnt, sidx, sb, s0)
                        for midx, mb, mcnt, _, sidx, sb in jobs)

                def keep():
                    return subcnts

                subcnts = lax.cond(c % SUPC == 0, rebuild, keep)
                pltpu.make_async_copy(tab.at[:, pl.ds(0, CK)], bufs[par],
                                      semc[par]).wait()
                for j, (_, _, _, dst, sidx, sb) in enumerate(jobs):
                    pending = process(bufs[par], lo + c * CK, CK, sidx, sb,
                                      subcnts[j], dst, pending)
                nxt = c + 2

                @pl.when(nxt < NCHK)
                def _():
                    r0 = pl.multiple_of(lo + nxt * CK, 128)
                    pltpu.async_copy(tab.at[:, pl.ds(r0, CK)], bufs[par],
                                     semc[par])
            return subcnts + (pending,)

        carry = lax.fori_loop(
            0, NCHK // 2, chunkpair,
            (jnp.int32(0),) * len(jobs) + (pend0,))
        *subcnts, pending = carry
        pltpu.make_async_copy(tab.at[:, pl.ds(0, CK)], buf0, sem0).wait()
        for j, (_, _, _, dst, sidx, sb) in enumerate(jobs):
            pending = process(buf0, lo + (NCHK - 1) * CK, CK, sidx, sb,
                              subcnts[j], dst, pending)
        return pending

    pend = scan_table(utab, ((mu, mbu, ucnt, uemb, s0i, s0b),),
                      jnp.int32(0))
    pend = scan_table(itab, ((mp, mbp, pcnt, pemb, s0i, s0b),
                             (mn, mbn, ncnt, nemb, s1i, s1b)), pend)

    def final_drain(pcount):
        def drain(i, c2):
            pltpu.make_async_copy(stag.at[pl.ds(0, L)],
                                  nemb.at[pl.ds(B, L)], sems).wait()
            return c2

        lax.fori_loop(0, pcount, drain, 0)

    @pl.when(wid == NW - 1)
    def _():
        pltpu.sync_copy(utail, tbuf)
        p2 = process(tbuf, SCAN, TAIL, mu, mbu, ucnt, uemb, pend)
        pltpu.sync_copy(itail, tbuf)
        p2 = process(tbuf, SCAN, TAIL, mp, mbp, pcnt, pemb, p2)
        p2 = process(tbuf, SCAN, TAIL, mn, mbn, ncnt, nemb, p2)
        final_drain(p2)

    @pl.when(wid != NW - 1)
    def _():
        final_drain(pend)


@functools.partial(
    pl.kernel,
    mesh=_mesh,
    compiler_params=pltpu.CompilerParams(needs_layout_passes=False),
    out_type=(
        jax.ShapeDtypeStruct((B,), jnp.float32),
        jax.ShapeDtypeStruct((B,), jnp.float32),
    ),
    scratch_types=[
        pltpu.VMEM((BPW // 4, ROWW), jnp.float32),
        pltpu.VMEM((BPW // 4, ROWW), jnp.float32),
        pltpu.VMEM((BPW // 4, ROWW), jnp.float32),
        pltpu.VMEM((BPW,), jnp.float32),
        pltpu.VMEM((BPW,), jnp.float32),
        pltpu.SemaphoreType.DMA,
    ],
)
def _dot_kernel(uemb, pemb, nemb, pout_hbm, nout_hbm,
                ubuf, pbuf, nbuf, pout, nout, sem):
    wid = lax.axis_index("s") * NC + lax.axis_index("c")
    base = wid * BPW
    SUB = BPW // 4

    def subchunk(s, carry):
        b0 = base + s * SUB
        cu = pltpu.async_copy(uemb.at[pl.ds(b0, SUB)], ubuf, sem)
        cp = pltpu.async_copy(pemb.at[pl.ds(b0, SUB)], pbuf, sem)
        cn = pltpu.async_copy(nemb.at[pl.ds(b0, SUB)], nbuf, sem)
        cu.wait(); cp.wait(); cn.wait()

        def group(g, carry2):
            row0 = g * L
            ridx = row0 + lax.iota(jnp.int32, L)
            accp = jnp.zeros((L,), jnp.float32)
            accn = jnp.zeros((L,), jnp.float32)
            for d in range(D):
                cidx = jnp.full((L,), d, jnp.int32)
                uv = plsc.load_gather(ubuf, [ridx, cidx])
                pv = plsc.load_gather(pbuf, [ridx, cidx])
                nv = plsc.load_gather(nbuf, [ridx, cidx])
                accp = accp + uv * pv
                accn = accn + uv * nv
            pout[pl.ds(s * SUB + row0, L)] = accp
            nout[pl.ds(s * SUB + row0, L)] = accn
            return carry2

        lax.fori_loop(0, SUB // L, group, 0)
        return carry

    lax.fori_loop(0, 4, subchunk, 0)
    pltpu.sync_copy(pout, pout_hbm.at[pl.ds(base, BPW)])
    pltpu.sync_copy(nout, nout_hbm.at[pl.ds(base, BPW)])


def kernel(users, positive_items, negative_items, user_table, item_table):
    u = users.astype(jnp.int32)
    p = positive_items.astype(jnp.int32)
    n = negative_items.astype(jnp.int32)
    ut = user_table.T
    it = item_table.T
    utail = ut[:, SCAN:]
    itail = it[:, SCAN:]
    uemb, pemb, nemb = _scan_kernel(u, p, n, ut, it, utail, itail)
    return _dot_kernel(uemb, pemb, nemb)
